# baseline (speedup 1.0000x reference)
import functools

import jax
import jax.numpy as jnp
from jax import lax
from jax.experimental import pallas as pl
from jax.experimental.pallas import tpu as pltpu
from jax.experimental.pallas import tpu_sc as plsc

N_NODES = 10000
D = 128
DH = 64
NPAD = 10112
DUMMY = 10048
NS = 16
NCH = 160
B = 128
DEGW = 16
ROWS_PER_TILE = NPAD // NS
GRP = 4
NSLOT = 2 * GRP
NB = NCH // NSLOT


def _sc_aggregate(xh, src3, dst3):
    mesh = plsc.VectorSubcoreMesh(core_axis_name="c", subcore_axis_name="s")

    @functools.partial(
        pl.kernel,
        mesh=mesh,
        compiler_params=pltpu.CompilerParams(use_tc_tiling_on_sc=False),
        out_type=[
            jax.ShapeDtypeStruct((2, NPAD, DH), jnp.float32),
            jax.ShapeDtypeStruct((2, NPAD, DEGW), jnp.float32),
        ],
        scratch_types=[
            pltpu.VMEM((2, NSLOT, B), jnp.int32),
            pltpu.VMEM((2, NSLOT, B), jnp.int32),
            pltpu.VMEM((NSLOT, B, DH), jnp.float32),
            pltpu.VMEM((B, DEGW), jnp.float32),
            pltpu.VMEM((B, DEGW), jnp.float32),
            pltpu.VMEM_SHARED((NPAD, DH), jnp.float32),
            pltpu.VMEM_SHARED((NPAD, DEGW), jnp.float32),
            [pltpu.SemaphoreType.DMA] * 2,
            [pltpu.SemaphoreType.DMA] * 2,
            pltpu.SemaphoreType.DMA,
        ],
    )
    def agg_kernel(x_hbm, src_hbm, dst_hbm, agg_out, deg_out,
                   sidx_v, didx_v, rows_v, one_v, z16_v,
                   agg_sh, deg_sh, gsems, isems, dsem):
        c = lax.axis_index("c")
        s = lax.axis_index("s")

        zeros16 = jnp.zeros((16,), jnp.float32)
        onehot = jnp.where(lax.iota(jnp.int32, 16) == 0,
                           jnp.float32(1.0), jnp.float32(0.0))

        def fill_body(i, _):
            for j in range(DH // 16):
                rows_v[0, i, pl.ds(j * 16, 16)] = zeros16
            one_v[i, :] = onehot
            z16_v[i, :] = zeros16
            return 0
        lax.fori_loop(0, B, fill_body, 0)

        zrows = rows_v.at[0]
        base = s * ROWS_PER_TILE
        for k in range(4):
            pltpu.sync_copy(zrows, agg_sh.at[pl.ds(base + k * B, B)])
            pltpu.sync_copy(z16_v, deg_sh.at[pl.ds(base + k * B, B)])
        rem = ROWS_PER_TILE - 4 * B
        pltpu.sync_copy(zrows.at[pl.ds(0, rem)],
                        agg_sh.at[pl.ds(base + 4 * B, rem)])
        pltpu.sync_copy(z16_v.at[pl.ds(0, rem)],
                        deg_sh.at[pl.ds(base + 4 * B, rem)])

        plsc.subcore_barrier()

        def idx_copies(blk, ib):
            return (
                pltpu.make_async_copy(
                    src_hbm.at[s].at[pl.ds(blk * NSLOT, NSLOT)],
                    sidx_v.at[ib], isems[ib]),
                pltpu.make_async_copy(
                    dst_hbm.at[s].at[pl.ds(blk * NSLOT, NSLOT)],
                    didx_v.at[ib], isems[ib]),
            )

        def gather(ib, row, slot, bank):
            return pltpu.make_async_copy(
                x_hbm.at[c].at[sidx_v.at[ib].at[row]],
                rows_v.at[slot], gsems[bank])

        def deg(ib, row):
            return pltpu.make_async_copy(
                one_v, deg_sh.at[didx_v.at[ib].at[row]], dsem)

        for cp in idx_copies(0, 0):
            cp.start()
        for cp in idx_copies(0, 0):
            cp.wait()
        for cp in idx_copies(1, 1):
            cp.start()
        for t in range(GRP):
            gather(0, t, t, 0).start()
        for t in range(GRP):
            gather(0, GRP + t, GRP + t, 1).start()

        def step(kk, _):
            j = kk * 2 * NSLOT

            def half(ib, joff, more_guard, prefetch_guard, pf_blk_off):
                @pl.when(more_guard)
                def _():
                    for cp in idx_copies(0, 1 - ib):
                        cp.wait()
                for bank in range(2):
                    for t in range(GRP):
                        gather(ib, bank * GRP + t, bank * GRP + t, bank).wait()
                    for t in range(GRP):
                        row = bank * GRP + t
                        pltpu.sync_copy(rows_v.at[row],
                                        agg_sh.at[didx_v.at[ib].at[row]],
                                        add=True)

                        @pl.when(c == row % 2)
                        def _():
                            pltpu.sync_copy(
                                one_v, deg_sh.at[didx_v.at[ib].at[row]],
                                add=True)

                    @pl.when(more_guard)
                    def _():
                        for t in range(GRP):
                            row = bank * GRP + t
                            gather(1 - ib, row, row, bank).start()
                @pl.when(prefetch_guard)
                def _():
                    for cp in idx_copies(j // NSLOT + pf_blk_off, ib):
                        cp.start()

            half(0, 0,
                 more_guard=(j + NSLOT < NCH),
                 prefetch_guard=(j + 2 * NSLOT < NCH), pf_blk_off=2)
            half(1, NSLOT,
                 more_guard=(j + 2 * NSLOT < NCH),
                 prefetch_guard=(j + 3 * NSLOT < NCH), pf_blk_off=3)
            return 0

        lax.fori_loop(0, NB // 2, step, 0)

        plsc.subcore_barrier()

        pltpu.sync_copy(agg_sh.at[pl.ds(base, ROWS_PER_TILE)],
                        agg_out.at[c, pl.ds(base, ROWS_PER_TILE)])
        pltpu.sync_copy(deg_sh.at[pl.ds(base, ROWS_PER_TILE)],
                        deg_out.at[c, pl.ds(base, ROWS_PER_TILE)])

    return agg_kernel(xh, src3, dst3)


def _tc_body(aggp_ref, degp_ref, w1_ref, b1_ref, w2_ref, b2_ref, out_ref):
    a = jnp.concatenate([aggp_ref[0], aggp_ref[1]], axis=1)
    d = degp_ref[0] + degp_ref[1]
    dsum = jnp.sum(d, axis=1, keepdims=True)
    a = a / jnp.maximum(dsum, 1.0)
    h = jnp.dot(a, w1_ref[...], preferred_element_type=jnp.float32) + b1_ref[...]
    h = jnp.maximum(h, 0.0)
    lg = jnp.dot(h, w2_ref[...], preferred_element_type=jnp.float32) + b2_ref[...]
    m = jnp.max(lg, axis=1, keepdims=True)
    out_ref[...] = (lg - m) - jnp.log(
        jnp.sum(jnp.exp(lg - m), axis=1, keepdims=True))


def _tc_epilogue(aggp, degp, W1, b1, W2, b2):
    R = 1000
    return pl.pallas_call(
        _tc_body,
        grid=(N_NODES // R,),
        in_specs=[
            pl.BlockSpec((2, R, DH), lambda i: (0, i, 0)),
            pl.BlockSpec((2, R, DEGW), lambda i: (0, i, 0)),
            pl.BlockSpec((D, D), lambda i: (0, 0)),
            pl.BlockSpec((1, D), lambda i: (0, 0)),
            pl.BlockSpec((D, 2), lambda i: (0, 0)),
            pl.BlockSpec((1, 2), lambda i: (0, 0)),
        ],
        out_specs=pl.BlockSpec((R, 2), lambda i: (i, 0)),
        out_shape=jax.ShapeDtypeStruct((N_NODES, 2), jnp.float32),
    )(aggp, degp, W1, b1, W2, b2)


def kernel(x, edge_index, W1, b1, W2, b2):
    src = edge_index[0].astype(jnp.int32)
    dst = edge_index[1].astype(jnp.int32)
    n_edges = src.shape[0]
    pad = NS * NCH * B - n_edges
    src3 = jnp.concatenate([src, jnp.zeros((pad,), jnp.int32)]).reshape(NS, NCH, B)
    dst3 = jnp.concatenate([dst, jnp.full((pad,), DUMMY, jnp.int32)]).reshape(NS, NCH, B)
    xh = x.reshape(N_NODES, 2, DH).transpose(1, 0, 2)
    aggp, degp = _sc_aggregate(xh, src3, dst3)
    return _tc_epilogue(aggp, degp, W1,
                        b1.reshape(1, D), W2, b2.reshape(1, 2))


# speedup vs baseline: 1.7232x; 1.7232x over previous
import functools

import jax
import jax.numpy as jnp
from jax import lax
from jax.experimental import pallas as pl
from jax.experimental.pallas import tpu as pltpu
from jax.experimental.pallas import tpu_sc as plsc

N_NODES = 10000
D = 128
DH = 64
NPAD = 10112
DUMMY = 10048
NS = 16
NCH = 157
B = 128
DEGW = 16
ROWS_PER_TILE = NPAD // NS
NSLOT = 4
NFULL = (NCH // NSLOT) * NSLOT
NTAIL = NCH - NFULL


def _sc_aggregate(xh, src3, dst3):
    mesh = plsc.VectorSubcoreMesh(core_axis_name="c", subcore_axis_name="s")

    @functools.partial(
        pl.kernel,
        mesh=mesh,
        compiler_params=pltpu.CompilerParams(use_tc_tiling_on_sc=False),
        out_type=[
            jax.ShapeDtypeStruct((2, NPAD, DH), jnp.float32),
            jax.ShapeDtypeStruct((2, NPAD, DEGW), jnp.float32),
        ],
        scratch_types=[
            pltpu.VMEM((NCH, B), jnp.int32),
            pltpu.VMEM((NCH, B), jnp.int32),
            pltpu.VMEM((NSLOT, B, DH), jnp.float32),
            pltpu.VMEM((B, DEGW), jnp.float32),
            pltpu.VMEM((B, DEGW), jnp.float32),
            pltpu.VMEM_SHARED((NPAD, DH), jnp.float32),
            pltpu.VMEM_SHARED((NPAD, DEGW), jnp.float32),
            [pltpu.SemaphoreType.DMA] * NSLOT,
        ],
    )
    def agg_kernel(x_hbm, src_hbm, dst_hbm, agg_out, deg_out,
                   src_v, dst_v, rows_v, one_v, z16_v,
                   agg_sh, deg_sh, gsems):
        c = lax.axis_index("c")
        s = lax.axis_index("s")

        zeros16 = jnp.zeros((16,), jnp.float32)
        onehot = jnp.where(lax.iota(jnp.int32, 16) == 0,
                           jnp.float32(1.0), jnp.float32(0.0))

        def fill_body(i, _):
            for j in range(DH // 16):
                rows_v[0, i, pl.ds(j * 16, 16)] = zeros16
            one_v[i, :] = onehot
            z16_v[i, :] = zeros16
            return 0
        lax.fori_loop(0, B, fill_body, 0)

        zrows = rows_v.at[0]
        base = s * ROWS_PER_TILE
        for k in range(4):
            pltpu.sync_copy(zrows, agg_sh.at[pl.ds(base + k * B, B)])
            pltpu.sync_copy(z16_v, deg_sh.at[pl.ds(base + k * B, B)])
        rem = ROWS_PER_TILE - 4 * B
        pltpu.sync_copy(zrows.at[pl.ds(0, rem)],
                        agg_sh.at[pl.ds(base + 4 * B, rem)])
        pltpu.sync_copy(z16_v.at[pl.ds(0, rem)],
                        deg_sh.at[pl.ds(base + 4 * B, rem)])

        pltpu.sync_copy(src_hbm.at[s], src_v)
        pltpu.sync_copy(dst_hbm.at[s], dst_v)

        plsc.subcore_barrier()

        def gather(j, slot):
            return pltpu.make_async_copy(
                x_hbm.at[c].at[src_v.at[j]], rows_v.at[slot], gsems[slot])

        def scatters(j, slot, do_deg):
            pltpu.sync_copy(rows_v.at[slot], agg_sh.at[dst_v.at[j]], add=True)

            @pl.when(c == do_deg)
            def _():
                pltpu.sync_copy(one_v, deg_sh.at[dst_v.at[j]], add=True)

        for t in range(NSLOT):
            gather(t, t).start()

        def body(jj, _):
            j = jj * NSLOT
            for t in range(NSLOT):
                gather(j + t, t).wait()
                scatters(j + t, t, t % 2)

                @pl.when(j + NSLOT + t < NCH)
                def _():
                    gather(j + NSLOT + t, t).start()
            return 0
        lax.fori_loop(0, NFULL // NSLOT, body, 0)

        for t in range(NTAIL):
            gather(NFULL + t, t).wait()
            scatters(NFULL + t, t, t % 2)

        plsc.subcore_barrier()

        pltpu.sync_copy(agg_sh.at[pl.ds(base, ROWS_PER_TILE)],
                        agg_out.at[c, pl.ds(base, ROWS_PER_TILE)])
        pltpu.sync_copy(deg_sh.at[pl.ds(base, ROWS_PER_TILE)],
                        deg_out.at[c, pl.ds(base, ROWS_PER_TILE)])

    return agg_kernel(xh, src3, dst3)


def _tc_body(aggp_ref, degp_ref, w1_ref, b1_ref, w2_ref, b2_ref, out_ref):
    a = jnp.concatenate([aggp_ref[0], aggp_ref[1]], axis=1)
    d = degp_ref[0] + degp_ref[1]
    dsum = jnp.sum(d, axis=1, keepdims=True)
    a = a / jnp.maximum(dsum, 1.0)
    h = jnp.dot(a, w1_ref[...], preferred_element_type=jnp.float32) + b1_ref[...]
    h = jnp.maximum(h, 0.0)
    lg = jnp.dot(h, w2_ref[...], preferred_element_type=jnp.float32) + b2_ref[...]
    m = jnp.max(lg, axis=1, keepdims=True)
    out_ref[...] = (lg - m) - jnp.log(
        jnp.sum(jnp.exp(lg - m), axis=1, keepdims=True))


def _tc_epilogue(aggp, degp, W1, b1, W2, b2):
    R = 1000
    return pl.pallas_call(
        _tc_body,
        grid=(N_NODES // R,),
        in_specs=[
            pl.BlockSpec((2, R, DH), lambda i: (0, i, 0)),
            pl.BlockSpec((2, R, DEGW), lambda i: (0, i, 0)),
            pl.BlockSpec((D, D), lambda i: (0, 0)),
            pl.BlockSpec((1, D), lambda i: (0, 0)),
            pl.BlockSpec((D, 2), lambda i: (0, 0)),
            pl.BlockSpec((1, 2), lambda i: (0, 0)),
        ],
        out_specs=pl.BlockSpec((R, 2), lambda i: (i, 0)),
        out_shape=jax.ShapeDtypeStruct((N_NODES, 2), jnp.float32),
    )(aggp, degp, W1, b1, W2, b2)


def kernel(x, edge_index, W1, b1, W2, b2):
    src = edge_index[0].astype(jnp.int32)
    dst = edge_index[1].astype(jnp.int32)
    n_edges = src.shape[0]
    pad = NS * NCH * B - n_edges
    src3 = jnp.concatenate([src, jnp.zeros((pad,), jnp.int32)]).reshape(NS, NCH, B)
    dst3 = jnp.concatenate([dst, jnp.full((pad,), DUMMY, jnp.int32)]).reshape(NS, NCH, B)
    xh = x.reshape(N_NODES, 2, DH).transpose(1, 0, 2)
    aggp, degp = _sc_aggregate(xh, src3, dst3)
    return _tc_epilogue(aggp, degp, W1,
                        b1.reshape(1, D), W2, b2.reshape(1, 2))


# speedup vs baseline: 1.7297x; 1.0038x over previous
import functools

import jax
import jax.numpy as jnp
from jax import lax
from jax.experimental import pallas as pl
from jax.experimental.pallas import tpu as pltpu
from jax.experimental.pallas import tpu_sc as plsc

N_NODES = 10000
D = 128
DH = 64
NPAD = 10112
DUMMY = 10048
NS = 16
NCH = 157
B = 128
DEGW = 16
ROWS_PER_TILE = NPAD // NS
NSLOT = 4
NFULL = (NCH // NSLOT) * NSLOT
NTAIL = NCH - NFULL


def _sc_aggregate(xh, src3, dst3):
    mesh = plsc.VectorSubcoreMesh(core_axis_name="c", subcore_axis_name="s")

    @functools.partial(
        pl.kernel,
        mesh=mesh,
        compiler_params=pltpu.CompilerParams(use_tc_tiling_on_sc=False),
        out_type=[
            jax.ShapeDtypeStruct((2, NPAD, DH), jnp.float32),
            jax.ShapeDtypeStruct((2, NPAD, DEGW), jnp.float32),
        ],
        scratch_types=[
            pltpu.VMEM((NCH, B), jnp.int32),
            pltpu.VMEM((NCH, B), jnp.int32),
            pltpu.VMEM((NSLOT, B, DH), jnp.float32),
            pltpu.VMEM((B, DEGW), jnp.float32),
            pltpu.VMEM((B, DEGW), jnp.float32),
            pltpu.VMEM_SHARED((NPAD, DH), jnp.float32),
            pltpu.VMEM_SHARED((NPAD, DEGW), jnp.float32),
            [pltpu.SemaphoreType.DMA] * NSLOT,
        ],
    )
    def agg_kernel(x_hbm, src_hbm, dst_hbm, agg_out, deg_out,
                   src_v, dst_v, rows_v, one_v, z16_v,
                   agg_sh, deg_sh, gsems):
        c = lax.axis_index("c")
        s = lax.axis_index("s")

        zeros16 = jnp.zeros((16,), jnp.float32)
        onehot = jnp.where(lax.iota(jnp.int32, 16) == 0,
                           jnp.float32(1.0), jnp.float32(0.0))

        def fill_body(i, _):
            for j in range(DH // 16):
                rows_v[0, i, pl.ds(j * 16, 16)] = zeros16
            one_v[i, :] = onehot
            z16_v[i, :] = zeros16
            return 0
        lax.fori_loop(0, B, fill_body, 0)

        zrows = rows_v.at[0]
        base = s * ROWS_PER_TILE
        for k in range(4):
            pltpu.sync_copy(zrows, agg_sh.at[pl.ds(base + k * B, B)])
            pltpu.sync_copy(z16_v, deg_sh.at[pl.ds(base + k * B, B)])
        rem = ROWS_PER_TILE - 4 * B
        pltpu.sync_copy(zrows.at[pl.ds(0, rem)],
                        agg_sh.at[pl.ds(base + 4 * B, rem)])
        pltpu.sync_copy(z16_v.at[pl.ds(0, rem)],
                        deg_sh.at[pl.ds(base + 4 * B, rem)])

        pltpu.sync_copy(src_hbm.at[c].at[s], src_v)
        pltpu.sync_copy(dst_hbm.at[s], dst_v)

        plsc.subcore_barrier()

        def gather(j, slot):
            return pltpu.make_async_copy(
                x_hbm.at[src_v.at[j]], rows_v.at[slot], gsems[slot])

        def scatters(j, slot, do_deg):
            pltpu.sync_copy(rows_v.at[slot], agg_sh.at[dst_v.at[j]], add=True)

            @pl.when(c == do_deg)
            def _():
                pltpu.sync_copy(one_v, deg_sh.at[dst_v.at[j]], add=True)

        for t in range(NSLOT):
            gather(t, t).start()

        def body(jj, _):
            j = jj * NSLOT
            for t in range(NSLOT):
                gather(j + t, t).wait()
                scatters(j + t, t, t % 2)

                @pl.when(j + NSLOT + t < NCH)
                def _():
                    gather(j + NSLOT + t, t).start()
            return 0
        lax.fori_loop(0, NFULL // NSLOT, body, 0)

        for t in range(NTAIL):
            gather(NFULL + t, t).wait()
            scatters(NFULL + t, t, t % 2)

        plsc.subcore_barrier()

        pltpu.sync_copy(agg_sh.at[pl.ds(base, ROWS_PER_TILE)],
                        agg_out.at[c, pl.ds(base, ROWS_PER_TILE)])
        pltpu.sync_copy(deg_sh.at[pl.ds(base, ROWS_PER_TILE)],
                        deg_out.at[c, pl.ds(base, ROWS_PER_TILE)])

    return agg_kernel(xh, src3, dst3)


def _tc_body(aggp_ref, degp_ref, w1_ref, b1_ref, w2_ref, b2_ref, out_ref):
    a = jnp.concatenate([aggp_ref[0], aggp_ref[1]], axis=1)
    d = degp_ref[0] + degp_ref[1]
    dsum = jnp.sum(d, axis=1, keepdims=True)
    a = a / jnp.maximum(dsum, 1.0)
    h = jnp.dot(a, w1_ref[...], preferred_element_type=jnp.float32) + b1_ref[...]
    h = jnp.maximum(h, 0.0)
    lg = jnp.dot(h, w2_ref[...], preferred_element_type=jnp.float32) + b2_ref[...]
    m = jnp.max(lg, axis=1, keepdims=True)
    out_ref[...] = (lg - m) - jnp.log(
        jnp.sum(jnp.exp(lg - m), axis=1, keepdims=True))


def _tc_epilogue(aggp, degp, W1, b1, W2, b2):
    R = 1000
    return pl.pallas_call(
        _tc_body,
        grid=(N_NODES // R,),
        in_specs=[
            pl.BlockSpec((2, R, DH), lambda i: (0, i, 0)),
            pl.BlockSpec((2, R, DEGW), lambda i: (0, i, 0)),
            pl.BlockSpec((D, D), lambda i: (0, 0)),
            pl.BlockSpec((1, D), lambda i: (0, 0)),
            pl.BlockSpec((D, 2), lambda i: (0, 0)),
            pl.BlockSpec((1, 2), lambda i: (0, 0)),
        ],
        out_specs=pl.BlockSpec((R, 2), lambda i: (i, 0)),
        out_shape=jax.ShapeDtypeStruct((N_NODES, 2), jnp.float32),
    )(aggp, degp, W1, b1, W2, b2)


def kernel(x, edge_index, W1, b1, W2, b2):
    src = edge_index[0].astype(jnp.int32)
    dst = edge_index[1].astype(jnp.int32)
    n_edges = src.shape[0]
    pad = NS * NCH * B - n_edges
    src2 = jnp.concatenate([2 * src, jnp.zeros((pad,), jnp.int32)])
    src3 = jnp.stack([src2, src2 + 1]).reshape(2, NS, NCH, B)
    dst3 = jnp.concatenate([dst, jnp.full((pad,), DUMMY, jnp.int32)]).reshape(NS, NCH, B)
    xh = x.reshape(2 * N_NODES, DH)
    aggp, degp = _sc_aggregate(xh, src3, dst3)
    return _tc_epilogue(aggp, degp, W1,
                        b1.reshape(1, D), W2, b2.reshape(1, 2))
